# Initial kernel scaffold; baseline (speedup 1.0000x reference)
import jax
import jax.numpy as jnp
from jax.experimental import pallas as pl


def kernel(indices, offsets, table, W, b):
    raise NotImplementedError("write your pallas kernel here")


# speedup vs baseline: 17.5049x; 17.5049x over previous
import functools

import jax
import jax.numpy as jnp
from jax import lax
from jax.experimental import pallas as pl
from jax.experimental.pallas import tpu as pltpu
from jax.experimental.pallas import tpu_sc as plsc

N = 819200
D = 64
VOCAB = 1000000

BP = 1024
VBLK = (VOCAB + 2 * BP - 1) // (2 * BP)
VPAD = VBLK * BP

_info = plsc.get_sparse_core_info()
NC, NS, L = _info.num_cores, _info.num_subcores, _info.num_lanes
NW = NC * NS

CHUNK = 128
ROWS_PER_W = N // NW
CHUNKS_PER_W = ROWS_PER_W // CHUNK
G = 8
GROUP = G * CHUNK
STEPS = CHUNKS_PER_W // G


def _relayout_body(xt_ref, eye_ref, o_ref):
    dn = (((0,), (0,)), ((), ()))
    y0 = lax.dot_general(xt_ref[:, 0:BP], eye_ref[...], dn,
                         preferred_element_type=jnp.float32)
    y1 = lax.dot_general(xt_ref[:, BP:2 * BP], eye_ref[...], dn,
                         preferred_element_type=jnp.float32)
    o_ref[:, 0:D] = y0
    o_ref[:, D:2 * D] = y1


def _tc_relayout(tableT, eye):
    return pl.pallas_call(
        _relayout_body,
        out_shape=jax.ShapeDtypeStruct((VPAD, 2 * D), jnp.float32),
        grid=(VBLK,),
        in_specs=[
            pl.BlockSpec((D, 2 * BP), lambda j: (0, j)),
            pl.BlockSpec((D, D), lambda j: (0, 0)),
        ],
        out_specs=pl.BlockSpec((BP, 2 * D), lambda j: (j, 0)),
    )(tableT, eye)


def _sc_gather(table_lin, idx3):
    mesh = plsc.VectorSubcoreMesh(core_axis_name="c", subcore_axis_name="s")

    @functools.partial(
        pl.kernel,
        mesh=mesh,
        out_type=jax.ShapeDtypeStruct((N // 2, 2 * D), jnp.float32),
        compiler_params=pltpu.CompilerParams(use_tc_tiling_on_sc=False),
        scratch_types=[
            pltpu.VMEM((CHUNKS_PER_W, CHUNK), jnp.int32),
            pltpu.VMEM((GROUP, D), jnp.float32),
            pltpu.SemaphoreType.DMA,
        ],
    )
    def gather_kernel(table_hbm, idx_hbm, out_hbm, idx_v, rows_v, sem):
        wid = lax.axis_index("s") * NC + lax.axis_index("c")
        pltpu.sync_copy(idx_hbm.at[wid], idx_v)

        def body(i, carry):
            base_chunk = i * G
            copies = [
                pltpu.async_copy(
                    table_hbm.at[idx_v.at[base_chunk + j]],
                    rows_v.at[pl.ds(j * CHUNK, CHUNK)],
                    sem,
                )
                for j in range(G)
            ]
            for c in copies:
                c.wait()
            g = wid * STEPS + i
            pltpu.sync_copy(
                rows_v,
                out_hbm.at[pl.ds((g // 2) * BP, BP),
                           pl.ds((g % 2) * D, D)],
            )
            return carry

        lax.fori_loop(0, STEPS, body, 0)

    return gather_kernel(table_lin, idx3)


def _mm_body(x_ref, w_ref, b_ref, o_ref):
    dn = (((1,), (1,)), ((), ()))
    y0 = lax.dot_general(w_ref[...], x_ref[:, 0:D], dn,
                         preferred_element_type=jnp.float32)
    y1 = lax.dot_general(w_ref[...], x_ref[:, D:2 * D], dn,
                         preferred_element_type=jnp.float32)
    o_ref[:, 0:BP] = y0 + b_ref[...]
    o_ref[:, BP:2 * BP] = y1 + b_ref[...]


def _tc_matmul(x2, W, b2):
    return pl.pallas_call(
        _mm_body,
        out_shape=jax.ShapeDtypeStruct((D, N), jnp.float32),
        grid=(N // (2 * BP),),
        in_specs=[
            pl.BlockSpec((BP, 2 * D), lambda j: (j, 0)),
            pl.BlockSpec((D, D), lambda j: (0, 0)),
            pl.BlockSpec((D, 1), lambda j: (0, 0)),
        ],
        out_specs=pl.BlockSpec((D, 2 * BP), lambda j: (0, j)),
    )(x2, W, b2)


def kernel(indices, offsets, table, W, b):
    del offsets
    blk = indices // (2 * BP)
    rem = indices % (2 * BP)
    idx2 = (blk * BP + (rem % BP)) * 2 + rem // BP
    idx3 = idx2.reshape(NW, CHUNKS_PER_W, CHUNK)
    eye = jnp.eye(D, dtype=jnp.float32)
    table2 = _tc_relayout(table.T, eye)
    table_lin = table2.reshape(2 * VPAD, D)
    x2 = _sc_gather(table_lin, idx3)
    out_t = _tc_matmul(x2, W, b.reshape(D, 1))
    return out_t.T


# speedup vs baseline: 18.8935x; 1.0793x over previous
import functools

import jax
import jax.numpy as jnp
from jax import lax
from jax.experimental import pallas as pl
from jax.experimental.pallas import tpu as pltpu
from jax.experimental.pallas import tpu_sc as plsc

N = 819200
D = 64
VOCAB = 1000000

BP = 1024
VBLK = (VOCAB + 2 * BP - 1) // (2 * BP)
VPAD = VBLK * BP

_info = plsc.get_sparse_core_info()
NC, NS, L = _info.num_cores, _info.num_subcores, _info.num_lanes
NW = NC * NS

CHUNK = 128
ROWS_PER_W = N // NW
CHUNKS_PER_W = ROWS_PER_W // CHUNK
G = 8
GROUP = G * CHUNK
STEPS = CHUNKS_PER_W // G


def _relayout_body(x0_ref, x1_ref, eye_ref, o_ref):
    xcat = jnp.concatenate([x0_ref[...], x1_ref[...]], axis=0)
    o_ref[...] = lax.dot_general(
        xcat, eye_ref[...], (((0,), (0,)), ((), ())),
        preferred_element_type=jnp.float32,
    )


def _tc_relayout(tableT, eye2):
    return pl.pallas_call(
        _relayout_body,
        out_shape=jax.ShapeDtypeStruct((VPAD, 2 * D), jnp.float32),
        grid=(VBLK,),
        in_specs=[
            pl.BlockSpec((D, BP), lambda j: (0, 2 * j)),
            pl.BlockSpec(
                (D, BP),
                lambda j: (0, jnp.minimum(2 * j + 1, VOCAB // BP - 1)),
            ),
            pl.BlockSpec((2 * D, 2 * D), lambda j: (0, 0)),
        ],
        out_specs=pl.BlockSpec((BP, 2 * D), lambda j: (j, 0)),
    )(tableT, tableT, eye2)


def _sc_gather(table_lin, idx3):
    mesh = plsc.VectorSubcoreMesh(core_axis_name="c", subcore_axis_name="s")

    @functools.partial(
        pl.kernel,
        mesh=mesh,
        out_type=jax.ShapeDtypeStruct((N // 2, 2 * D), jnp.float32),
        compiler_params=pltpu.CompilerParams(use_tc_tiling_on_sc=False),
        scratch_types=[
            pltpu.VMEM((CHUNKS_PER_W, CHUNK), jnp.int32),
            pltpu.VMEM((GROUP, D), jnp.float32),
            pltpu.SemaphoreType.DMA,
        ],
    )
    def gather_kernel(table_hbm, idx_hbm, out_hbm, idx_v, rows_v, sem):
        wid = lax.axis_index("s") * NC + lax.axis_index("c")
        pltpu.sync_copy(idx_hbm.at[wid], idx_v)

        def body(i, carry):
            base_chunk = i * G
            copies = [
                pltpu.async_copy(
                    table_hbm.at[idx_v.at[base_chunk + j]],
                    rows_v.at[pl.ds(j * CHUNK, CHUNK)],
                    sem,
                )
                for j in range(G)
            ]
            for c in copies:
                c.wait()
            g = wid * STEPS + i
            pltpu.sync_copy(
                rows_v,
                out_hbm.at[pl.ds((g // 2) * BP, BP),
                           pl.ds((g % 2) * D, D)],
            )
            return carry

        lax.fori_loop(0, STEPS, body, 0)

    return gather_kernel(table_lin, idx3)


def _mm_body(x_ref, wblk_ref, b_ref, o_ref):
    zz = lax.dot_general(
        wblk_ref[...], x_ref[...], (((1,), (1,)), ((), ())),
        preferred_element_type=jnp.float32,
    )
    o_ref[:, 0:BP] = zz[0:D, :] + b_ref[...]
    o_ref[:, BP:2 * BP] = zz[D:2 * D, :] + b_ref[...]


def _tc_matmul(x2, Wblk, b2):
    return pl.pallas_call(
        _mm_body,
        out_shape=jax.ShapeDtypeStruct((D, N), jnp.float32),
        grid=(N // (2 * BP),),
        in_specs=[
            pl.BlockSpec((BP, 2 * D), lambda j: (j, 0)),
            pl.BlockSpec((2 * D, 2 * D), lambda j: (0, 0)),
            pl.BlockSpec((D, 1), lambda j: (0, 0)),
        ],
        out_specs=pl.BlockSpec((D, 2 * BP), lambda j: (0, j)),
    )(x2, Wblk, b2)


def kernel(indices, offsets, table, W, b):
    del offsets
    blk = indices // (2 * BP)
    rem = indices % (2 * BP)
    idx2 = (blk * BP + (rem % BP)) * 2 + rem // BP
    idx3 = idx2.reshape(NW, CHUNKS_PER_W, CHUNK)
    eye2 = jnp.eye(2 * D, dtype=jnp.float32)
    wblk = jnp.kron(jnp.eye(2, dtype=jnp.float32), W)
    table2 = _tc_relayout(table.T, eye2)
    table_lin = table2.reshape(2 * VPAD, D)
    x2 = _sc_gather(table_lin, idx3)
    out_t = _tc_matmul(x2, wblk, b.reshape(D, 1))
    return out_t.T


# speedup vs baseline: 25.2704x; 1.3375x over previous
import functools

import jax
import jax.numpy as jnp
from jax import lax
from jax.experimental import pallas as pl
from jax.experimental.pallas import tpu as pltpu
from jax.experimental.pallas import tpu_sc as plsc

N = 819200
D = 64
VOCAB = 1000000

BP = 2048
VBLK = (VOCAB + 2 * BP - 1) // (2 * BP)
VPAD = VBLK * BP

_info = plsc.get_sparse_core_info()
NC, NS, L = _info.num_cores, _info.num_subcores, _info.num_lanes
NW = NC * NS

CHUNK = 128
ROWS_PER_W = N // NW
CHUNKS_PER_W = ROWS_PER_W // CHUNK
G = 8
GROUP = G * CHUNK
STEPS = CHUNKS_PER_W // G


def _relayout_body(x0_ref, x1_ref, eye_ref, o_ref):
    xcat = jnp.concatenate([x0_ref[...], x1_ref[...]], axis=0)
    o_ref[...] = lax.dot_general(
        xcat, eye_ref[...], (((0,), (0,)), ((), ())),
        preferred_element_type=jnp.float32,
    )


def _tc_relayout(tableT, eye2):
    return pl.pallas_call(
        _relayout_body,
        out_shape=jax.ShapeDtypeStruct((VPAD, 2 * D), jnp.float32),
        grid=(VBLK,),
        in_specs=[
            pl.BlockSpec((D, BP), lambda j: (0, 2 * j)),
            pl.BlockSpec(
                (D, BP),
                lambda j: (0, jnp.minimum(2 * j + 1, VOCAB // BP - 1)),
            ),
            pl.BlockSpec((2 * D, 2 * D), lambda j: (0, 0)),
        ],
        out_specs=pl.BlockSpec((BP, 2 * D), lambda j: (j, 0)),
    )(tableT, tableT, eye2)


def _sc_gather(table_lin, idx3):
    mesh = plsc.VectorSubcoreMesh(core_axis_name="c", subcore_axis_name="s")

    @functools.partial(
        pl.kernel,
        mesh=mesh,
        out_type=jax.ShapeDtypeStruct((N // 2, 2 * D), jnp.float32),
        compiler_params=pltpu.CompilerParams(use_tc_tiling_on_sc=False),
        scratch_types=[
            pltpu.VMEM((CHUNKS_PER_W, CHUNK), jnp.int32),
            pltpu.VMEM((GROUP, D), jnp.float32),
            pltpu.SemaphoreType.DMA,
        ],
    )
    def gather_kernel(table_hbm, idx_hbm, out_hbm, idx_v, rows_v, sem):
        wid = lax.axis_index("s") * NC + lax.axis_index("c")
        pltpu.sync_copy(idx_hbm.at[wid], idx_v)

        def body(i, carry):
            base_chunk = i * G
            copies = [
                pltpu.async_copy(
                    table_hbm.at[idx_v.at[base_chunk + j]],
                    rows_v.at[pl.ds(j * CHUNK, CHUNK)],
                    sem,
                )
                for j in range(G)
            ]
            for c in copies:
                c.wait()
            g = wid * STEPS + i
            c0 = g * GROUP
            blk = c0 // (2 * BP)
            rem = c0 % (2 * BP)
            pltpu.sync_copy(
                rows_v,
                out_hbm.at[pl.ds(blk * BP + rem % BP, GROUP),
                           pl.ds((rem // BP) * D, D)],
            )
            return carry

        lax.fori_loop(0, STEPS, body, 0)

    return gather_kernel(table_lin, idx3)


def _mm_body(x_ref, wblk_ref, b_ref, o_ref):
    zz = lax.dot_general(
        wblk_ref[...], x_ref[...], (((1,), (1,)), ((), ())),
        preferred_element_type=jnp.float32,
    )
    o_ref[:, 0:BP] = zz[0:D, :] + b_ref[...]
    o_ref[:, BP:2 * BP] = zz[D:2 * D, :] + b_ref[...]


def _tc_matmul(x2, Wblk, b2):
    return pl.pallas_call(
        _mm_body,
        out_shape=jax.ShapeDtypeStruct((D, N), jnp.float32),
        grid=(N // (2 * BP),),
        in_specs=[
            pl.BlockSpec((BP, 2 * D), lambda j: (j, 0)),
            pl.BlockSpec((2 * D, 2 * D), lambda j: (0, 0)),
            pl.BlockSpec((D, 1), lambda j: (0, 0)),
        ],
        out_specs=pl.BlockSpec((D, 2 * BP), lambda j: (0, j)),
    )(x2, Wblk, b2)


def kernel(indices, offsets, table, W, b):
    del offsets
    blk = indices // (2 * BP)
    rem = indices % (2 * BP)
    idx2 = (blk * BP + (rem % BP)) * 2 + rem // BP
    idx3 = idx2.reshape(NW, CHUNKS_PER_W, CHUNK)
    eye2 = jnp.eye(2 * D, dtype=jnp.float32)
    wblk = jnp.kron(jnp.eye(2, dtype=jnp.float32), W)
    table2 = _tc_relayout(table.T, eye2)
    table_lin = table2.reshape(2 * VPAD, D)
    x2 = _sc_gather(table_lin, idx3)
    out_t = _tc_matmul(x2, wblk, b.reshape(D, 1))
    return out_t.T


# speedup vs baseline: 32.3231x; 1.2791x over previous
import functools

import jax
import jax.numpy as jnp
from jax import lax
from jax.experimental import pallas as pl
from jax.experimental.pallas import tpu as pltpu
from jax.experimental.pallas import tpu_sc as plsc

N = 819200
D = 64
VOCAB = 1000000

BP = 4096
VBLK = (VOCAB + 2 * BP - 1) // (2 * BP)
VPAD = VBLK * BP

_info = plsc.get_sparse_core_info()
NC, NS, L = _info.num_cores, _info.num_subcores, _info.num_lanes
NW = NC * NS

CHUNK = 128
ROWS_PER_W = N // NW
CHUNKS_PER_W = ROWS_PER_W // CHUNK
G = 8
GROUP = G * CHUNK
STEPS = CHUNKS_PER_W // G


def _relayout_body(x0_ref, x1_ref, eye_ref, o_ref):
    xcat = jnp.concatenate([x0_ref[...], x1_ref[...]], axis=0)
    o_ref[...] = lax.dot_general(
        xcat, eye_ref[...], (((0,), (0,)), ((), ())),
        preferred_element_type=jnp.float32,
    )


def _tc_relayout(tableT, eye2):
    return pl.pallas_call(
        _relayout_body,
        out_shape=jax.ShapeDtypeStruct((VPAD, 2 * D), jnp.float32),
        grid=(VBLK,),
        in_specs=[
            pl.BlockSpec((D, BP), lambda j: (0, 2 * j)),
            pl.BlockSpec(
                (D, BP),
                lambda j: (0, jnp.minimum(2 * j + 1, VOCAB // BP - 1)),
            ),
            pl.BlockSpec((2 * D, 2 * D), lambda j: (0, 0)),
        ],
        out_specs=pl.BlockSpec((BP, 2 * D), lambda j: (j, 0)),
    )(tableT, tableT, eye2)


def _sc_gather(table_lin, idx3):
    mesh = plsc.VectorSubcoreMesh(core_axis_name="c", subcore_axis_name="s")

    @functools.partial(
        pl.kernel,
        mesh=mesh,
        out_type=jax.ShapeDtypeStruct((N // 2, 2 * D), jnp.float32),
        compiler_params=pltpu.CompilerParams(use_tc_tiling_on_sc=False),
        scratch_types=[
            pltpu.VMEM((CHUNKS_PER_W, CHUNK), jnp.int32),
            pltpu.VMEM((GROUP, D), jnp.float32),
            pltpu.SemaphoreType.DMA,
        ],
    )
    def gather_kernel(table_hbm, idx_hbm, out_hbm, idx_v, rows_v, sem):
        wid = lax.axis_index("s") * NC + lax.axis_index("c")
        pltpu.sync_copy(idx_hbm.at[wid], idx_v)

        def body(i, carry):
            base_chunk = i * G
            copies = [
                pltpu.async_copy(
                    table_hbm.at[idx_v.at[base_chunk + j]],
                    rows_v.at[pl.ds(j * CHUNK, CHUNK)],
                    sem,
                )
                for j in range(G)
            ]
            for c in copies:
                c.wait()
            g = wid * STEPS + i
            c0 = g * GROUP
            blk = c0 // (2 * BP)
            rem = c0 % (2 * BP)
            pltpu.sync_copy(
                rows_v,
                out_hbm.at[pl.ds(blk * BP + rem % BP, GROUP),
                           pl.ds((rem // BP) * D, D)],
            )
            return carry

        lax.fori_loop(0, STEPS, body, 0)

    return gather_kernel(table_lin, idx3)


def _mm_body(x_ref, wblk_ref, b_ref, o_ref):
    zz = lax.dot_general(
        wblk_ref[...], x_ref[...], (((1,), (1,)), ((), ())),
        preferred_element_type=jnp.float32,
    )
    o_ref[:, 0:BP] = zz[0:D, :] + b_ref[...]
    o_ref[:, BP:2 * BP] = zz[D:2 * D, :] + b_ref[...]


def _tc_matmul(x2, Wblk, b2):
    return pl.pallas_call(
        _mm_body,
        out_shape=jax.ShapeDtypeStruct((D, N), jnp.float32),
        grid=(N // (2 * BP),),
        in_specs=[
            pl.BlockSpec((BP, 2 * D), lambda j: (j, 0)),
            pl.BlockSpec((2 * D, 2 * D), lambda j: (0, 0)),
            pl.BlockSpec((D, 1), lambda j: (0, 0)),
        ],
        out_specs=pl.BlockSpec((D, 2 * BP), lambda j: (0, j)),
    )(x2, Wblk, b2)


def kernel(indices, offsets, table, W, b):
    del offsets
    blk = indices // (2 * BP)
    rem = indices % (2 * BP)
    idx2 = (blk * BP + (rem % BP)) * 2 + rem // BP
    idx3 = idx2.reshape(NW, CHUNKS_PER_W, CHUNK)
    eye2 = jnp.eye(2 * D, dtype=jnp.float32)
    wblk = jnp.kron(jnp.eye(2, dtype=jnp.float32), W)
    table2 = _tc_relayout(table.T, eye2)
    table_lin = table2.reshape(2 * VPAD, D)
    x2 = _sc_gather(table_lin, idx3)
    out_t = _tc_matmul(x2, wblk, b.reshape(D, 1))
    return out_t.T


# speedup vs baseline: 35.7645x; 1.1065x over previous
import functools

import jax
import jax.numpy as jnp
from jax import lax
from jax.experimental import pallas as pl
from jax.experimental.pallas import tpu as pltpu
from jax.experimental.pallas import tpu_sc as plsc

N = 819200
D = 64
VOCAB = 1000000

BP = 8192
VBLK = (VOCAB + 2 * BP - 1) // (2 * BP)
VPAD = VBLK * BP

_info = plsc.get_sparse_core_info()
NC, NS, L = _info.num_cores, _info.num_subcores, _info.num_lanes
NW = NC * NS

CHUNK = 128
ROWS_PER_W = N // NW
CHUNKS_PER_W = ROWS_PER_W // CHUNK
G = 8
GROUP = G * CHUNK
STEPS = CHUNKS_PER_W // G


def _relayout_body(x0_ref, x1_ref, eye_ref, o_ref):
    xcat = jnp.concatenate([x0_ref[...], x1_ref[...]], axis=0)
    o_ref[...] = lax.dot_general(
        xcat, eye_ref[...], (((0,), (0,)), ((), ())),
        preferred_element_type=jnp.float32,
    )


def _tc_relayout(tableT, eye2):
    return pl.pallas_call(
        _relayout_body,
        out_shape=jax.ShapeDtypeStruct((VPAD, 2 * D), jnp.float32),
        grid=(VBLK,),
        in_specs=[
            pl.BlockSpec((D, BP), lambda j: (0, 2 * j)),
            pl.BlockSpec(
                (D, BP),
                lambda j: (0, jnp.minimum(2 * j + 1, VOCAB // BP - 1)),
            ),
            pl.BlockSpec((2 * D, 2 * D), lambda j: (0, 0)),
        ],
        out_specs=pl.BlockSpec((BP, 2 * D), lambda j: (j, 0)),
    )(tableT, tableT, eye2)


def _sc_gather(table_lin, idx3):
    mesh = plsc.VectorSubcoreMesh(core_axis_name="c", subcore_axis_name="s")

    @functools.partial(
        pl.kernel,
        mesh=mesh,
        out_type=jax.ShapeDtypeStruct((N // 2, 2 * D), jnp.float32),
        compiler_params=pltpu.CompilerParams(use_tc_tiling_on_sc=False),
        scratch_types=[
            pltpu.VMEM((CHUNKS_PER_W, CHUNK), jnp.int32),
            pltpu.VMEM((GROUP, D), jnp.float32),
            pltpu.SemaphoreType.DMA,
        ],
    )
    def gather_kernel(table_hbm, idx_hbm, out_hbm, idx_v, rows_v, sem):
        wid = lax.axis_index("s") * NC + lax.axis_index("c")
        pltpu.sync_copy(idx_hbm.at[wid], idx_v)

        def body(i, carry):
            base_chunk = i * G
            copies = [
                pltpu.async_copy(
                    table_hbm.at[idx_v.at[base_chunk + j]],
                    rows_v.at[pl.ds(j * CHUNK, CHUNK)],
                    sem,
                )
                for j in range(G)
            ]
            for c in copies:
                c.wait()
            g = wid * STEPS + i
            c0 = g * GROUP
            blk = c0 // (2 * BP)
            rem = c0 % (2 * BP)
            pltpu.sync_copy(
                rows_v,
                out_hbm.at[pl.ds(blk * BP + rem % BP, GROUP),
                           pl.ds((rem // BP) * D, D)],
            )
            return carry

        lax.fori_loop(0, STEPS, body, 0)

    return gather_kernel(table_lin, idx3)


def _mm_body(x_ref, wblk_ref, b_ref, o_ref):
    zz = lax.dot_general(
        wblk_ref[...], x_ref[...], (((1,), (1,)), ((), ())),
        preferred_element_type=jnp.float32,
    )
    o_ref[:, 0:BP] = zz[0:D, :] + b_ref[...]
    o_ref[:, BP:2 * BP] = zz[D:2 * D, :] + b_ref[...]


def _tc_matmul(x2, Wblk, b2):
    return pl.pallas_call(
        _mm_body,
        out_shape=jax.ShapeDtypeStruct((D, N), jnp.float32),
        grid=(N // (2 * BP),),
        in_specs=[
            pl.BlockSpec((BP, 2 * D), lambda j: (j, 0)),
            pl.BlockSpec((2 * D, 2 * D), lambda j: (0, 0)),
            pl.BlockSpec((D, 1), lambda j: (0, 0)),
        ],
        out_specs=pl.BlockSpec((D, 2 * BP), lambda j: (0, j)),
    )(x2, Wblk, b2)


def kernel(indices, offsets, table, W, b):
    del offsets
    blk = indices // (2 * BP)
    rem = indices % (2 * BP)
    idx2 = (blk * BP + (rem % BP)) * 2 + rem // BP
    idx3 = idx2.reshape(NW, CHUNKS_PER_W, CHUNK)
    eye2 = jnp.eye(2 * D, dtype=jnp.float32)
    wblk = jnp.kron(jnp.eye(2, dtype=jnp.float32), W)
    table2 = _tc_relayout(table.T, eye2)
    table_lin = table2.reshape(2 * VPAD, D)
    x2 = _sc_gather(table_lin, idx3)
    out_t = _tc_matmul(x2, wblk, b.reshape(D, 1))
    return out_t.T
